# Initial kernel scaffold; baseline (speedup 1.0000x reference)
import jax
import jax.numpy as jnp
from jax.experimental import pallas as pl


def kernel(tensor):
    raise NotImplementedError("write your pallas kernel here")


# speedup vs baseline: 27.3268x; 27.3268x over previous
import jax
import jax.numpy as jnp
import numpy as np
from jax.experimental import pallas as pl

_PERCENT = 0.96
_CHUNK = 2048

_IMAX = np.int32(2**31 - 1)
_IMIN = np.int32(-(2**31))


def _skey(xb):
    bits = jax.lax.bitcast_convert_type(xb, jnp.int32)
    return bits ^ ((bits >> 31) & _IMAX)


def _pqp_kernel(x_ref, o_ref):
    nrows, ncols = x_ref.shape
    n = nrows * ncols
    loc = _PERCENT * (n - 1)
    k = int(loc)
    frac = jnp.float32(loc - k)
    nchunks = ncols // _CHUNK

    def chunk_keys(c):
        return _skey(x_ref[:, pl.ds(c * _CHUNK, _CHUNK)])

    def count_le(t):
        def body(c, acc):
            return acc + jnp.sum((chunk_keys(c) <= t).astype(jnp.int32))
        return jax.lax.fori_loop(0, nchunks, body, np.int32(0))

    rank1 = np.int32(k + 1)

    def bisect_body(_, lohi):
        lo, hi = lohi
        mid = (lo & hi) + ((lo ^ hi) >> 1)
        ge = count_le(mid) >= rank1
        return (jnp.where(ge, lo, mid), jnp.where(ge, mid, hi))

    _, k1 = jax.lax.fori_loop(0, 32, bisect_body, (_IMIN, _IMAX))

    c1 = count_le(k1)

    def min_gt_body(c, acc):
        sk = chunk_keys(c)
        return jnp.minimum(acc, jnp.min(jnp.where(sk > k1, sk, _IMAX)))

    k2_above = jax.lax.fori_loop(0, nchunks, min_gt_body, _IMAX)
    k2 = jnp.where(c1 >= rank1 + 1, k1, k2_above)

    def tofloat(sk):
        return jax.lax.bitcast_convert_type(
            sk ^ ((sk >> 31) & _IMAX), jnp.float32)

    vk = tofloat(k1)
    vk1 = tofloat(k2)
    tresh = vk + frac * (vk1 - vk)

    def write_body(c, carry):
        xb = x_ref[:, pl.ds(c * _CHUNK, _CHUNK)]
        o_ref[:, pl.ds(c * _CHUNK, _CHUNK)] = jnp.where(xb > tresh, vk, xb)
        return carry

    jax.lax.fori_loop(0, nchunks, write_body, 0)


def kernel(tensor):
    return pl.pallas_call(
        _pqp_kernel,
        out_shape=jax.ShapeDtypeStruct(tensor.shape, tensor.dtype),
    )(tensor)
